# baseline (speedup 1.0000x reference)
import jax
import jax.numpy as jnp
from jax.experimental import pallas as pl
from jax.experimental.pallas import tpu as pltpu

_N = 100000
_D = 128
_BLK = 2000


def _body(x_ref, ones_ref, c_ref, o_ref, acc_ref):
    i = pl.program_id(0)
    c = c_ref[...]
    z = c * (c - 2.0 * x_ref[...])
    mm = jax.lax.dot_general(
        z, ones_ref[...], (((1,), (0,)), ((), ())),
        preferred_element_type=jnp.float32,
    )

    @pl.when(i == 0)
    def _init():
        acc_ref[...] = mm

    @pl.when(i > 0)
    def _acc():
        acc_ref[...] = jnp.minimum(acc_ref[...], mm)

    @pl.when(i == pl.num_programs(0) - 1)
    def _fin():
        a = acc_ref[...]
        t = jnp.minimum(a[: _BLK // 2], a[_BLK // 2 :])
        t = jnp.minimum(t[: _BLK // 4], t[_BLK // 4 :])
        m = jnp.min(t)
        xv = x_ref[...]
        x2 = jnp.sum(xv * xv)
        o_ref[...] = jnp.sqrt(jnp.maximum(m + x2, 0.0)).reshape(1, 1)


def kernel(x, corpus):
    ones_mat = jnp.ones((_D, _D), dtype=jnp.float32)
    out = pl.pallas_call(
        _body,
        grid=(_N // _BLK,),
        in_specs=[
            pl.BlockSpec((1, _D), lambda i: (0, 0)),
            pl.BlockSpec((_D, _D), lambda i: (0, 0)),
            pl.BlockSpec((_BLK, _D), lambda i: (i, 0)),
        ],
        out_specs=pl.BlockSpec((1, 1), lambda i: (0, 0)),
        out_shape=jax.ShapeDtypeStruct((1, 1), jnp.float32),
        scratch_shapes=[pltpu.VMEM((_BLK, _D), jnp.float32)],
    )(x.reshape(1, _D), ones_mat, corpus)
    return out[0, 0]
